# baseline (speedup 1.0000x reference)
import functools

import jax
import jax.numpy as jnp
from jax import lax
from jax.experimental import pallas as pl
from jax.experimental.pallas import tpu as pltpu
from jax.experimental.pallas import tpu_sc as plsc

_NUM_CORES = 2
_NUM_SUBCORES = 16
_NUM_WORKERS = _NUM_CORES * _NUM_SUBCORES

_LANES = 16
_BLK = 128
_GRP = 5


def _embedding_lookup(ids, weight, n_rows, seq):
  _, dim = weight.shape
  b_per_w = n_rows // _NUM_WORKERS
  n_blk = b_per_w // _BLK
  blk_ids = _BLK * seq
  jbs = _BLK // _LANES
  mesh = plsc.VectorSubcoreMesh(core_axis_name="c", subcore_axis_name="s")

  @functools.partial(
      pl.kernel,
      out_type=jax.ShapeDtypeStruct((seq, dim, n_rows), jnp.float32),
      mesh=mesh,
      scratch_types=[
          pltpu.VMEM((blk_ids,), jnp.int32),
          pltpu.VMEM((blk_ids,), jnp.int32),
          pltpu.VMEM((2, _GRP * _BLK, dim), jnp.float32),
          pltpu.VMEM((_GRP, dim, _BLK + 1), jnp.float32),
          pltpu.SemaphoreType.DMA((2,)),
          pltpu.SemaphoreType.DMA((_GRP,)),
      ],
      compiler_params=pltpu.CompilerParams(
          use_tc_tiling_on_sc=False, needs_layout_passes=False),
  )
  def k(idx_hbm, table_hbm, out_hbm, idx_raw, idx_t, rows_v, t_v, gsem, osem):
    wid = lax.axis_index("s") * _NUM_CORES + lax.axis_index("c")
    lane_j = [
        jnp.arange(jb * _LANES, (jb + 1) * _LANES, dtype=jnp.int32)
        for jb in range(jbs)
    ]
    lane_f = [jnp.full((_LANES,), f, jnp.int32) for f in range(dim)]

    def gather_start(g, buf):
      return pltpu.async_copy(
          table_hbm.at[idx_t.at[pl.ds(g * (_GRP * _BLK), _GRP * _BLK)]],
          rows_v.at[buf], gsem.at[buf])

    def gather_wait(g, buf):
      pltpu.make_async_copy(
          table_hbm.at[idx_t.at[pl.ds(g * (_GRP * _BLK), _GRP * _BLK)]],
          rows_v.at[buf], gsem.at[buf]).wait()

    def store_start(s, base_b, ls):
      return pltpu.async_copy(
          t_v.at[ls, :, pl.ds(0, _BLK)],
          out_hbm.at[s, :, pl.ds(base_b, _BLK)], osem.at[ls])

    def store_wait(s, base_b, ls):
      pltpu.make_async_copy(
          t_v.at[ls, :, pl.ds(0, _BLK)],
          out_hbm.at[s, :, pl.ds(base_b, _BLK)],
          osem.at[ls]).wait()

    halves = dim // _LANES
    lane_h = [
        jnp.arange(h * _LANES, (h + 1) * _LANES, dtype=jnp.int32)
        for h in range(halves)
    ]

    def transpose(gbuf, ls):
      for j in range(_BLK):
        col_j = jnp.full((_LANES,), j, jnp.int32)
        for h in range(halves):
          vals = rows_v[gbuf, ls * _BLK + j, pl.ds(h * _LANES, _LANES)]
          plsc.store_scatter(t_v.at[ls], [lane_h[h], col_j], vals)

    def blk_body(t, _):
      base_b = wid * b_per_w + t * _BLK
      pltpu.sync_copy(idx_hbm.at[pl.ds(base_b * seq, blk_ids)], idx_raw)

      for s in range(seq):
        for jb in range(jbs):
          vals = plsc.load_gather(idx_raw, [lane_j[jb] * seq + s])
          idx_t[pl.ds(s * _BLK + jb * _LANES, _LANES)] = vals

      gather_start(0, 0)
      n_grp = seq // _GRP

      def grp_body(g, _):
        gbuf = lax.rem(g, 2)

        @pl.when(g < n_grp - 1)
        def _():
          gather_start(g + 1, 1 - gbuf)
        gather_wait(g, gbuf)
        for ls in range(_GRP):
          s = g * _GRP + ls

          @pl.when(g > 0)
          def _():
            store_wait(s - _GRP, base_b, ls)
          transpose(gbuf, ls)
          store_start(s, base_b, ls)
        return 0
      lax.fori_loop(0, n_grp, grp_body, 0)

      for ls in range(_GRP):
        store_wait(seq - _GRP + ls, base_b, ls)
      return 0

    lax.fori_loop(0, n_blk, blk_body, 0)

  out = k(ids, weight)
  return out


def kernel(token_ids, weight):
  n_rows, seq = token_ids.shape
  flat = token_ids.reshape(n_rows * seq).astype(jnp.int32)
  out = _embedding_lookup(flat, weight, n_rows, seq)
  return jnp.transpose(out, (2, 0, 1))


# speedup vs baseline: 1.0180x; 1.0180x over previous
import functools

import jax
import jax.numpy as jnp
from jax import lax
from jax.experimental import pallas as pl
from jax.experimental.pallas import tpu as pltpu
from jax.experimental.pallas import tpu_sc as plsc

_NUM_CORES = 2
_NUM_SUBCORES = 16
_NUM_WORKERS = _NUM_CORES * _NUM_SUBCORES

_LANES = 16
_BLK = 128


def _embedding_lookup(ids, weight, n_rows, seq):
  _, dim = weight.shape
  b_per_w = n_rows // _NUM_WORKERS
  n_blk = b_per_w // _BLK
  blk_ids = _BLK * seq
  jbs = _BLK // _LANES
  mesh = plsc.VectorSubcoreMesh(core_axis_name="c", subcore_axis_name="s")

  @functools.partial(
      pl.kernel,
      out_type=jax.ShapeDtypeStruct((seq, dim, n_rows), jnp.float32),
      mesh=mesh,
      scratch_types=[
          pltpu.VMEM((blk_ids,), jnp.int32),
          pltpu.VMEM((blk_ids,), jnp.int32),
          pltpu.VMEM((2, 2 * _BLK, dim), jnp.float32),
          pltpu.VMEM((2, dim, _BLK + 1), jnp.float32),
          pltpu.SemaphoreType.DMA((2,)),
          pltpu.SemaphoreType.DMA((2,)),
      ],
      compiler_params=pltpu.CompilerParams(
          use_tc_tiling_on_sc=False, needs_layout_passes=False),
  )
  def k(idx_hbm, table_hbm, out_hbm, idx_raw, idx_t, rows_v, t_v, gsem, osem):
    wid = lax.axis_index("s") * _NUM_CORES + lax.axis_index("c")
    lane_j = [
        jnp.arange(jb * _LANES, (jb + 1) * _LANES, dtype=jnp.int32)
        for jb in range(jbs)
    ]
    lane_f = [jnp.full((_LANES,), f, jnp.int32) for f in range(dim)]

    def gather_start(p, buf):
      return pltpu.async_copy(
          table_hbm.at[idx_t.at[pl.ds(p * (2 * _BLK), 2 * _BLK)]],
          rows_v.at[buf], gsem.at[buf])

    def gather_wait(p, buf):
      pltpu.make_async_copy(
          table_hbm.at[idx_t.at[pl.ds(p * (2 * _BLK), 2 * _BLK)]],
          rows_v.at[buf], gsem.at[buf]).wait()

    def store_start(s, base_b, buf):
      return pltpu.async_copy(
          t_v.at[buf, :, pl.ds(0, _BLK)],
          out_hbm.at[s, :, pl.ds(base_b, _BLK)], osem.at[buf])

    def store_wait(s, base_b, buf):
      pltpu.make_async_copy(
          t_v.at[buf, :, pl.ds(0, _BLK)],
          out_hbm.at[s, :, pl.ds(base_b, _BLK)],
          osem.at[buf]).wait()

    halves = dim // _LANES
    lane_h = [
        jnp.arange(h * _LANES, (h + 1) * _LANES, dtype=jnp.int32)
        for h in range(halves)
    ]

    def transpose(gbuf, u):
      for j in range(_BLK):
        col_j = jnp.full((_LANES,), j, jnp.int32)
        for h in range(halves):
          vals = rows_v[gbuf, u * _BLK + j, pl.ds(h * _LANES, _LANES)]
          plsc.store_scatter(t_v.at[u], [lane_h[h], col_j], vals)

    def blk_body(t, _):
      base_b = wid * b_per_w + t * _BLK
      pltpu.sync_copy(idx_hbm.at[pl.ds(base_b * seq, blk_ids)], idx_raw)

      for s in range(seq):
        for jb in range(jbs):
          vals = plsc.load_gather(idx_raw, [lane_j[jb] * seq + s])
          idx_t[pl.ds(s * _BLK + jb * _LANES, _LANES)] = vals

      gather_start(0, 0)
      n_pair = seq // 2

      def pair_body2(i4, _):
        for v in range(2):
          p = 2 * i4 + v
          gbuf = v
          ngbuf = 1 - v
          if v == 0:
            gather_start(p + 1, ngbuf)
          else:
            @pl.when(i4 < n_pair // 2 - 1)
            def _():
              gather_start(p + 1, ngbuf)
          gather_wait(p, gbuf)
          for u in range(2):
            s = 2 * p + u

            @pl.when(p > 0)
            def _():
              store_wait(s - 2, base_b, u)
            transpose(gbuf, u)
            store_start(s, base_b, u)
        return 0
      lax.fori_loop(0, n_pair // 2, pair_body2, 0)

      store_wait(seq - 2, base_b, 0)
      store_wait(seq - 1, base_b, 1)
      return 0

    lax.fori_loop(0, n_blk, blk_body, 0)

  out = k(ids, weight)
  return out


def kernel(token_ids, weight):
  n_rows, seq = token_ids.shape
  flat = token_ids.reshape(n_rows * seq).astype(jnp.int32)
  out = _embedding_lookup(flat, weight, n_rows, seq)
  return jnp.transpose(out, (2, 0, 1))


# speedup vs baseline: 1.0576x; 1.0389x over previous
import functools

import jax
import jax.numpy as jnp
from jax import lax
from jax.experimental import pallas as pl
from jax.experimental.pallas import tpu as pltpu
from jax.experimental.pallas import tpu_sc as plsc

_NUM_CORES = 2
_NUM_SUBCORES = 16
_NUM_WORKERS = _NUM_CORES * _NUM_SUBCORES

_LANES = 16
_BLK = 128


def _embedding_lookup(ids, weight, n_rows, seq):
  _, dim = weight.shape
  b_per_w = n_rows // _NUM_WORKERS
  n_blk = b_per_w // _BLK
  blk_ids = _BLK * seq
  jbs = _BLK // _LANES
  mesh = plsc.VectorSubcoreMesh(core_axis_name="c", subcore_axis_name="s")

  @functools.partial(
      pl.kernel,
      out_type=jax.ShapeDtypeStruct((seq, dim, n_rows), jnp.float32),
      mesh=mesh,
      scratch_types=[
          pltpu.VMEM((blk_ids,), jnp.int32),
          pltpu.VMEM((blk_ids,), jnp.int32),
          pltpu.VMEM((2, _BLK, dim), jnp.float32),
          pltpu.VMEM((2, dim, _BLK + 1), jnp.float32),
          pltpu.SemaphoreType.DMA((2,)),
          pltpu.SemaphoreType.DMA((2,)),
      ],
      compiler_params=pltpu.CompilerParams(
          use_tc_tiling_on_sc=False, needs_layout_passes=False),
  )
  def k(idx_hbm, table_hbm, out_hbm, idx_raw, idx_t, rows_v, t_v, gsem, osem):
    wid = lax.axis_index("s") * _NUM_CORES + lax.axis_index("c")
    lane_j = [
        jnp.arange(jb * _LANES, (jb + 1) * _LANES, dtype=jnp.int32)
        for jb in range(jbs)
    ]
    lane_f = [jnp.full((_LANES,), f, jnp.int32) for f in range(dim)]

    def gather_start(s, buf):
      return pltpu.async_copy(
          table_hbm.at[idx_t.at[pl.ds(s * _BLK, _BLK)]], rows_v.at[buf],
          gsem.at[buf])

    def gather_wait(s, buf):
      pltpu.make_async_copy(
          table_hbm.at[idx_t.at[pl.ds(s * _BLK, _BLK)]], rows_v.at[buf],
          gsem.at[buf]).wait()

    def store_start(s, base_b, buf):
      return pltpu.async_copy(
          t_v.at[buf, :, pl.ds(0, _BLK)],
          out_hbm.at[s, :, pl.ds(base_b, _BLK)], osem.at[buf])

    def store_wait(s, base_b, buf):
      pltpu.make_async_copy(
          t_v.at[buf, :, pl.ds(0, _BLK)],
          out_hbm.at[s, :, pl.ds(base_b, _BLK)],
          osem.at[buf]).wait()

    halves = dim // _LANES
    lane_h = [
        jnp.arange(h * _LANES, (h + 1) * _LANES, dtype=jnp.int32)
        for h in range(halves)
    ]

    def transpose(buf):
      for j in range(_BLK):
        col_j = jnp.full((_LANES,), j, jnp.int32)
        for h in range(halves):
          vals = rows_v[buf, j, pl.ds(h * _LANES, _LANES)]
          plsc.store_scatter(t_v.at[buf], [lane_h[h], col_j], vals)

    def blk_body(t, _):
      base_b = wid * b_per_w + t * _BLK
      pltpu.sync_copy(idx_hbm.at[pl.ds(base_b * seq, blk_ids)], idx_raw)

      for s in range(seq):
        for jb in range(jbs):
          vals = plsc.load_gather(idx_raw, [lane_j[jb] * seq + s])
          idx_t[pl.ds(s * _BLK + jb * _LANES, _LANES)] = vals

      gather_start(0, 0)

      def pair_body(i2, _):
        for u in range(2):
          s = 2 * i2 + u
          buf = u
          nbuf = 1 - u
          if u == 0:
            gather_start(s + 1, nbuf)
          else:
            @pl.when(i2 < seq // 2 - 1)
            def _():
              gather_start(s + 1, nbuf)
          gather_wait(s, buf)

          @pl.when(i2 > 0)
          def _():
            store_wait(s - 2, base_b, buf)
          transpose(buf)
          store_start(s, base_b, buf)
        return 0
      lax.fori_loop(0, seq // 2, pair_body, 0)

      store_wait(seq - 2, base_b, 0)
      store_wait(seq - 1, base_b, 1)
      return 0

    lax.fori_loop(0, n_blk, blk_body, 0)

  out = k(ids, weight)
  return out


def kernel(token_ids, weight):
  n_rows, seq = token_ids.shape
  flat = token_ids.reshape(n_rows * seq).astype(jnp.int32)
  out = _embedding_lookup(flat, weight, n_rows, seq)
  return jnp.transpose(out, (2, 0, 1))
